# Initial kernel scaffold; baseline (speedup 1.0000x reference)
import jax
import jax.numpy as jnp
from jax.experimental import pallas as pl


def kernel(query_feats, db_text, db_images, db_audio, db_video, W_text, top_k):
    raise NotImplementedError("write your pallas kernel here")


# speedup vs baseline: 2.7172x; 2.7172x over previous
import functools

import jax
import jax.numpy as jnp
import numpy as np
from jax import lax
from jax.experimental import pallas as pl
from jax.experimental.pallas import tpu as pltpu
from jax.experimental.pallas import tpu_sc as plsc

Q = 256
D_IN = 256
D = 128
K = 100000
TOPK = 16
BK = 2048
NB = 49
KP = NB * BK
G = 32
NGB = BK // G
NG = NB * NGB
NM = 4
ITEMS = NM * Q
NEGF = np.float32(-1.0e30)
NEGINF = np.float32(-3.0e38)
BIGI = np.int32(2**31 - 1)
NC = 2
NS = 16
NW = NC * NS
IPW = ITEMS // NW
NGV = NG // 16
L2N = TOPK * G
L2V = L2N // 16


def _tc_body(qf_ref, w_ref, dbt_ref, dbi_ref, dba_ref, dbv_ref,
             sims_ref, gm_ref, qn_ref):
    j = pl.program_id(0)

    @pl.when(j == 0)
    def _():
        q = lax.dot_general(qf_ref[...], w_ref[...],
                            (((1,), (0,)), ((), ())),
                            preferred_element_type=jnp.float32)
        nq = jnp.sqrt(jnp.sum(q * q, axis=1, keepdims=True))
        qn_ref[...] = q / (nq + 1e-8)

    q = qn_ref[...]
    col = j * BK + lax.broadcasted_iota(jnp.int32, (Q, BK), 1)
    valid = col < K
    for m, db_ref in enumerate((dbt_ref, dbi_ref, dba_ref, dbv_ref)):
        db = db_ref[...]
        nd = jnp.sqrt(jnp.sum(db * db, axis=1, keepdims=True))
        dbn = db / (nd + 1e-8)
        s = lax.dot_general(q, dbn, (((1,), (1,)), ((), ())),
                            preferred_element_type=jnp.float32)
        s = jnp.where(valid, s, NEGF)
        sims_ref[m] = s
        gm_ref[m, :, 0, 0, :] = jnp.max(s.reshape(Q, NGB, G), axis=2)


def _tc_stage(query_feats, W_text, db_text, db_images, db_audio, db_video):
    return pl.pallas_call(
        _tc_body,
        grid=(NB,),
        in_specs=[
            pl.BlockSpec((Q, D_IN), lambda j: (0, 0)),
            pl.BlockSpec((D_IN, D), lambda j: (0, 0)),
            pl.BlockSpec((BK, D), lambda j: (j, 0)),
            pl.BlockSpec((BK, D), lambda j: (j, 0)),
            pl.BlockSpec((BK, D), lambda j: (j, 0)),
            pl.BlockSpec((BK, D), lambda j: (j, 0)),
        ],
        out_specs=[
            pl.BlockSpec((NM, Q, BK), lambda j: (0, 0, j)),
            pl.BlockSpec((NM, Q, 1, 1, NGB), lambda j: (0, 0, j, 0, 0)),
        ],
        out_shape=[
            jax.ShapeDtypeStruct((NM, Q, KP), jnp.float32),
            jax.ShapeDtypeStruct((NM, Q, NB, 1, NGB), jnp.float32),
        ],
        scratch_shapes=[pltpu.VMEM((Q, D), jnp.float32)],
    )(query_feats, W_text, db_text, db_images, db_audio, db_video)


def _lane_best(i, carry, val_ref, id_ref):
    m, gi = carry
    v = val_ref[pl.ds(i * 16, 16)]
    g = id_ref[pl.ds(i * 16, 16)]
    upd = (v > m) | ((v == m) & (g < gi))
    return jnp.where(upd, v, m), jnp.where(upd, g, gi)


def _mask_out(i, _, val_ref, id_ref, vstar, istar):
    v = val_ref[pl.ds(i * 16, 16)]
    g = id_ref[pl.ds(i * 16, 16)]
    hit = (v == vstar) & (g == istar)
    val_ref[pl.ds(i * 16, 16)] = jnp.where(hit, NEGINF, v)
    return 0


def _select_round(val_ref, id_ref, ncv):
    m0 = jnp.full((16,), NEGINF, jnp.float32)
    g0 = jnp.full((16,), BIGI, jnp.int32)
    m, gi = lax.fori_loop(
        0, ncv, functools.partial(_lane_best, val_ref=val_ref, id_ref=id_ref),
        (m0, g0))
    vstar = jnp.max(m)
    istar = jnp.min(jnp.where(m == vstar, gi, BIGI))
    lax.fori_loop(
        0, ncv,
        functools.partial(_mask_out, val_ref=val_ref, id_ref=id_ref,
                          vstar=vstar, istar=istar), 0)
    return vstar, istar


def _compact(val_ref, id_ref, src_vec, id_vec, thresh, off):
    msk = src_vec >= thresh
    pos = off + plsc.cumsum(msk.astype(jnp.int32)) - 1
    plsc.store_scatter(val_ref, [pos], src_vec, mask=msk)
    plsc.store_scatter(id_ref, [pos], id_vec, mask=msk)
    return off + plsc.all_reduce_population_count(msk)


def _sc_item(item, gm_hbm, sims_hbm, vals_hbm, idx_hbm,
             gm_v, cval, cgid, l2_v, gsel_ref, outv, outi, sem, lanes):
    pltpu.sync_copy(gm_hbm.at[pl.ds(item * NG, NG)], gm_v)

    def maxbody(i, m):
        return jnp.maximum(m, gm_v[pl.ds(i * 16, 16)])
    t0 = jnp.min(lax.fori_loop(0, NGV, maxbody, jnp.full((16,), NEGINF,
                                                         jnp.float32)))

    def cbody(i, off):
        return _compact(cval, cgid, gm_v[pl.ds(i * 16, 16)],
                        i * 16 + lanes, t0, off)
    off = lax.fori_loop(0, NGV, cbody, jnp.zeros((16,), jnp.int32))
    cnt = jnp.max(off)
    plsc.store_scatter(cval, [cnt + lanes], jnp.full((16,), NEGINF,
                                                     jnp.float32))
    plsc.store_scatter(cgid, [cnt + lanes], jnp.full((16,), BIGI, jnp.int32))
    ncv = (cnt + 15) >> 4

    gsel = jnp.zeros((16,), jnp.int32)
    copies = []
    for r in range(TOPK):
        _, istar = _select_round(cval, cgid, ncv)
        gsel = jnp.where(lanes == r, istar, gsel)
        src = sims_hbm.at[pl.ds(item * KP + istar * G, G)]
        copies.append(pltpu.async_copy(src, l2_v.at[pl.ds(r * G, G)], sem))
    for c in copies:
        c.wait()
    gsel_ref[...] = gsel

    def max2(i, m):
        return jnp.maximum(m, l2_v[pl.ds(i * 16, 16)])
    t1 = jnp.min(lax.fori_loop(0, L2V, max2, jnp.full((16,), NEGINF,
                                                      jnp.float32)))

    def c2body(i, off2):
        p = i * 16 + lanes
        grp = plsc.load_gather(gsel_ref, [p >> 5])
        colv = (grp << 5) + (p & 31)
        return _compact(cval, cgid, l2_v[pl.ds(i * 16, 16)], colv, t1, off2)
    off2 = lax.fori_loop(0, L2V, c2body, jnp.zeros((16,), jnp.int32))
    cnt2 = jnp.max(off2)
    plsc.store_scatter(cval, [cnt2 + lanes], jnp.full((16,), NEGINF,
                                                      jnp.float32))
    plsc.store_scatter(cgid, [cnt2 + lanes], jnp.full((16,), BIGI, jnp.int32))
    ncv2 = (cnt2 + 15) >> 4

    ov = jnp.full((16,), 0.0, jnp.float32)
    oi = jnp.zeros((16,), jnp.int32)
    for r in range(TOPK):
        vstar, istar = _select_round(cval, cgid, ncv2)
        ov = jnp.where(lanes == r, vstar, ov)
        oi = jnp.where(lanes == r, istar, oi)
    outv[...] = ov
    outi[...] = oi
    pltpu.sync_copy(outv, vals_hbm.at[pl.ds(item * TOPK, TOPK)])
    pltpu.sync_copy(outi, idx_hbm.at[pl.ds(item * TOPK, TOPK)])
    return 0


def _sc_stage(gm_flat, sims_flat):
    mesh = plsc.VectorSubcoreMesh(core_axis_name="c", subcore_axis_name="s")

    @functools.partial(
        pl.kernel,
        out_type=[
            jax.ShapeDtypeStruct((ITEMS * TOPK,), jnp.float32),
            jax.ShapeDtypeStruct((ITEMS * TOPK,), jnp.int32),
        ],
        mesh=mesh,
        compiler_params=pltpu.CompilerParams(needs_layout_passes=False),
        scratch_types=[
            pltpu.VMEM((NG,), jnp.float32),
            pltpu.VMEM((NG + 16,), jnp.float32),
            pltpu.VMEM((NG + 16,), jnp.int32),
            pltpu.VMEM((L2N,), jnp.float32),
            pltpu.VMEM((16,), jnp.int32),
            pltpu.VMEM((16,), jnp.float32),
            pltpu.VMEM((16,), jnp.int32),
            pltpu.SemaphoreType.DMA,
        ],
    )
    def sc_kernel(gm_hbm, sims_hbm, vals_hbm, idx_hbm,
                  gm_v, cval, cgid, l2_v, gsel_ref, outv, outi, sem):
        wid = lax.axis_index("s") * NC + lax.axis_index("c")
        lanes = jnp.arange(16, dtype=jnp.int32)
        lax.fori_loop(
            0, IPW,
            lambda t, _: _sc_item(wid * IPW + t, gm_hbm, sims_hbm,
                                  vals_hbm, idx_hbm, gm_v, cval, cgid,
                                  l2_v, gsel_ref, outv, outi, sem, lanes),
            0)

    return sc_kernel(gm_flat, sims_flat)


def kernel(query_feats, db_text, db_images, db_audio, db_video, W_text, top_k):
    sims, gm = _tc_stage(query_feats, W_text,
                         db_text, db_images, db_audio, db_video)
    vals_flat, idx_flat = _sc_stage(gm.reshape(ITEMS * NG),
                                    sims.reshape(ITEMS * KP))
    vals = vals_flat.reshape(NM, Q, TOPK)
    idx = idx_flat.reshape(NM, Q, TOPK)
    return (vals[0], idx[0], vals[1], idx[1],
            vals[2], idx[2], vals[3], idx[3])


# speedup vs baseline: 6.1889x; 2.2776x over previous
import functools

import jax
import jax.numpy as jnp
import numpy as np
from jax import lax
from jax.experimental import pallas as pl
from jax.experimental.pallas import tpu as pltpu
from jax.experimental.pallas import tpu_sc as plsc

Q = 256
D_IN = 256
D = 128
K = 100000
TOPK = 16
BK = 2048
NB = 49
KP = NB * BK
G = 128
NGB = BK // G
NG = NB * NGB
NM = 4
ITEMS = NM * Q
NEGF = np.float32(-1.0e30)
NEGINF = np.float32(-3.0e38)
BIGI = np.int32(2**31 - 1)
Q8 = Q // 8
NT = KP // G
NC = 2
NS = 16
NW = NC * NS
IPW = ITEMS // NW
NGV = NG // 16
L2N = TOPK * G
L2V = L2N // 16


def _tc_body(qf_ref, w_ref, dbt_ref, dbi_ref, dba_ref, dbv_ref,
             sims_ref, gm_ref, qn_ref):
    j = pl.program_id(0)

    @pl.when(j == 0)
    def _():
        q = lax.dot_general(qf_ref[...], w_ref[...],
                            (((1,), (0,)), ((), ())),
                            preferred_element_type=jnp.float32)
        nq = jnp.sqrt(jnp.sum(q * q, axis=1, keepdims=True))
        qn_ref[...] = q / (nq + 1e-8)

    q = qn_ref[...]
    col = j * BK + lax.broadcasted_iota(jnp.int32, (Q, BK), 1)
    valid = col < K
    for m, db_ref in enumerate((dbt_ref, dbi_ref, dba_ref, dbv_ref)):
        db = db_ref[...]
        nd = jnp.sqrt(jnp.sum(db * db, axis=1, keepdims=True))
        dbn = db / (nd + 1e-8)
        s = lax.dot_general(q, dbn, (((1,), (1,)), ((), ())),
                            preferred_element_type=jnp.float32)
        s = jnp.where(valid, s, NEGF)
        for t in range(NGB):
            sims_ref[m, :, t] = s[:, t * G:(t + 1) * G].reshape(Q8, 8, G)
        gm_ref[m, :, 0, 0, :] = jnp.max(s.reshape(Q, NGB, G), axis=2)


def _tc_stage(query_feats, W_text, db_text, db_images, db_audio, db_video):
    return pl.pallas_call(
        _tc_body,
        grid=(NB,),
        in_specs=[
            pl.BlockSpec((Q, D_IN), lambda j: (0, 0)),
            pl.BlockSpec((D_IN, D), lambda j: (0, 0)),
            pl.BlockSpec((BK, D), lambda j: (j, 0)),
            pl.BlockSpec((BK, D), lambda j: (j, 0)),
            pl.BlockSpec((BK, D), lambda j: (j, 0)),
            pl.BlockSpec((BK, D), lambda j: (j, 0)),
        ],
        out_specs=[
            pl.BlockSpec((NM, Q8, NGB, 8, G), lambda j: (0, 0, j, 0, 0)),
            pl.BlockSpec((NM, Q, 1, 1, NGB), lambda j: (0, 0, j, 0, 0)),
        ],
        out_shape=[
            jax.ShapeDtypeStruct((NM, Q8, NT, 8, G), jnp.float32),
            jax.ShapeDtypeStruct((NM, Q, NB, 1, NGB), jnp.float32),
        ],
        scratch_shapes=[pltpu.VMEM((Q, D), jnp.float32)],
    )(query_feats, W_text, db_text, db_images, db_audio, db_video)


def _lane_best(i, carry, val_ref, id_ref):
    m, gi = carry
    v = val_ref[pl.ds(i * 16, 16)]
    g = id_ref[pl.ds(i * 16, 16)]
    upd = (v > m) | ((v == m) & (g < gi))
    return jnp.where(upd, v, m), jnp.where(upd, g, gi)


def _mask_out(i, _, val_ref, id_ref, vstar, istar):
    v = val_ref[pl.ds(i * 16, 16)]
    g = id_ref[pl.ds(i * 16, 16)]
    hit = (v == vstar) & (g == istar)
    val_ref[pl.ds(i * 16, 16)] = jnp.where(hit, NEGINF, v)
    return 0


def _select_round(val_ref, id_ref, ncv):
    m0 = jnp.full((16,), NEGINF, jnp.float32)
    g0 = jnp.full((16,), BIGI, jnp.int32)
    m, gi = lax.fori_loop(
        0, ncv, functools.partial(_lane_best, val_ref=val_ref, id_ref=id_ref),
        (m0, g0))
    vstar = jnp.max(m)
    istar = jnp.min(jnp.where(m == vstar, gi, BIGI))
    lax.fori_loop(
        0, ncv,
        functools.partial(_mask_out, val_ref=val_ref, id_ref=id_ref,
                          vstar=vstar, istar=istar), 0)
    return vstar, istar


def _compact(val_ref, id_ref, src_vec, id_vec, thresh, off):
    msk = src_vec >= thresh
    pos = off + plsc.cumsum(msk.astype(jnp.int32)) - 1
    plsc.store_scatter(val_ref, [pos], src_vec, mask=msk)
    plsc.store_scatter(id_ref, [pos], id_vec, mask=msk)
    return off + plsc.all_reduce_population_count(msk)


def _sc_item(item, gm_hbm, sims_hbm, vals_hbm, idx_hbm,
             gm_v, cval, cgid, l2_v, gsel_ref, outv, outi, sem, lanes):
    pltpu.sync_copy(gm_hbm.at[pl.ds(item * NG, NG)], gm_v)

    def maxbody(i, m):
        return jnp.maximum(m, gm_v[pl.ds(i * 16, 16)])
    t0 = jnp.min(lax.fori_loop(0, NGV, maxbody, jnp.full((16,), NEGINF,
                                                         jnp.float32)))

    def cbody(i, off):
        return _compact(cval, cgid, gm_v[pl.ds(i * 16, 16)],
                        i * 16 + lanes, t0, off)
    off = lax.fori_loop(0, NGV, cbody, jnp.zeros((16,), jnp.int32))
    cnt = jnp.max(off)
    plsc.store_scatter(cval, [cnt + lanes], jnp.full((16,), NEGINF,
                                                     jnp.float32))
    plsc.store_scatter(cgid, [cnt + lanes], jnp.full((16,), BIGI, jnp.int32))
    ncv = (cnt + 15) >> 4

    m_ix = item >> 8
    q_ix = item & (Q - 1)
    base = (m_ix * Q8 + (q_ix >> 3)) * (NT * 8 * G) + (q_ix & 7) * G
    gsel = jnp.zeros((16,), jnp.int32)
    copies = []
    for r in range(TOPK):
        _, istar = _select_round(cval, cgid, ncv)
        gsel = jnp.where(lanes == r, istar, gsel)
        src = sims_hbm.at[pl.ds(base + istar * (8 * G), G)]
        copies.append(pltpu.async_copy(src, l2_v.at[pl.ds(r * G, G)], sem))
    for c in copies:
        c.wait()
    gsel_ref[...] = gsel

    def max2(i, m):
        return jnp.maximum(m, l2_v[pl.ds(i * 16, 16)])
    t1 = jnp.min(lax.fori_loop(0, L2V, max2, jnp.full((16,), NEGINF,
                                                      jnp.float32)))

    def c2body(i, off2):
        p = i * 16 + lanes
        grp = plsc.load_gather(gsel_ref, [p >> 7])
        colv = (grp << 7) + (p & 127)
        return _compact(cval, cgid, l2_v[pl.ds(i * 16, 16)], colv, t1, off2)
    off2 = lax.fori_loop(0, L2V, c2body, jnp.zeros((16,), jnp.int32))
    cnt2 = jnp.max(off2)
    plsc.store_scatter(cval, [cnt2 + lanes], jnp.full((16,), NEGINF,
                                                      jnp.float32))
    plsc.store_scatter(cgid, [cnt2 + lanes], jnp.full((16,), BIGI, jnp.int32))
    ncv2 = (cnt2 + 15) >> 4

    ov = jnp.full((16,), 0.0, jnp.float32)
    oi = jnp.zeros((16,), jnp.int32)
    for r in range(TOPK):
        vstar, istar = _select_round(cval, cgid, ncv2)
        ov = jnp.where(lanes == r, vstar, ov)
        oi = jnp.where(lanes == r, istar, oi)
    outv[...] = ov
    outi[...] = oi
    pltpu.sync_copy(outv, vals_hbm.at[pl.ds(item * TOPK, TOPK)])
    pltpu.sync_copy(outi, idx_hbm.at[pl.ds(item * TOPK, TOPK)])
    return 0


def _sc_stage(gm_flat, sims_flat):
    mesh = plsc.VectorSubcoreMesh(core_axis_name="c", subcore_axis_name="s")

    @functools.partial(
        pl.kernel,
        out_type=[
            jax.ShapeDtypeStruct((ITEMS * TOPK,), jnp.float32),
            jax.ShapeDtypeStruct((ITEMS * TOPK,), jnp.int32),
        ],
        mesh=mesh,
        compiler_params=pltpu.CompilerParams(needs_layout_passes=False),
        scratch_types=[
            pltpu.VMEM((NG,), jnp.float32),
            pltpu.VMEM((NG + 16,), jnp.float32),
            pltpu.VMEM((NG + 16,), jnp.int32),
            pltpu.VMEM((L2N,), jnp.float32),
            pltpu.VMEM((16,), jnp.int32),
            pltpu.VMEM((16,), jnp.float32),
            pltpu.VMEM((16,), jnp.int32),
            pltpu.SemaphoreType.DMA,
        ],
    )
    def sc_kernel(gm_hbm, sims_hbm, vals_hbm, idx_hbm,
                  gm_v, cval, cgid, l2_v, gsel_ref, outv, outi, sem):
        wid = lax.axis_index("s") * NC + lax.axis_index("c")
        lanes = jnp.arange(16, dtype=jnp.int32)
        lax.fori_loop(
            0, IPW,
            lambda t, _: _sc_item(wid * IPW + t, gm_hbm, sims_hbm,
                                  vals_hbm, idx_hbm, gm_v, cval, cgid,
                                  l2_v, gsel_ref, outv, outi, sem, lanes),
            0)

    return sc_kernel(gm_flat, sims_flat)


def kernel(query_feats, db_text, db_images, db_audio, db_video, W_text, top_k):
    sims, gm = _tc_stage(query_feats, W_text,
                         db_text, db_images, db_audio, db_video)
    vals_flat, idx_flat = _sc_stage(gm.reshape(ITEMS * NG),
                                    sims.reshape(NM * Q8 * NT * 8 * G))
    vals = vals_flat.reshape(NM, Q, TOPK)
    idx = idx_flat.reshape(NM, Q, TOPK)
    return (vals[0], idx[0], vals[1], idx[1],
            vals[2], idx[2], vals[3], idx[3])


# speedup vs baseline: 7.5797x; 1.2247x over previous
import functools

import jax
import jax.numpy as jnp
import numpy as np
from jax import lax
from jax.experimental import pallas as pl
from jax.experimental.pallas import tpu as pltpu
from jax.experimental.pallas import tpu_sc as plsc

Q = 256
D_IN = 256
D = 128
K = 100000
TOPK = 16
BK = 2048
NB = 49
KP = NB * BK
G = 128
NGB = BK // G
NG = NB * NGB
NM = 4
ITEMS = NM * Q
NEGF = np.float32(-1.0e30)
NEGINF = np.float32(-3.0e38)
BIGI = np.int32(2**31 - 1)
Q8 = Q // 8
NT = KP // G
NC = 2
NS = 16
NW = NC * NS
NBUN = ITEMS // 8
BPW = NBUN // NW
GMB = NB * 8 * G
L2N = TOPK * G
L2V = L2N // 16
CBUF = L2N + 16


def _tc_body(qf_ref, w_ref, dbt_ref, dbi_ref, dba_ref, dbv_ref,
             sims_ref, gm_ref, qn_ref):
    j = pl.program_id(0)

    @pl.when(j == 0)
    def _():
        q = lax.dot_general(qf_ref[...], w_ref[...],
                            (((1,), (0,)), ((), ())),
                            preferred_element_type=jnp.float32)
        nq = jnp.sqrt(jnp.sum(q * q, axis=1, keepdims=True))
        qn_ref[...] = q / (nq + 1e-8)

    q = qn_ref[...]
    col = j * BK + lax.broadcasted_iota(jnp.int32, (Q, BK), 1)
    valid = col < K
    for m, db_ref in enumerate((dbt_ref, dbi_ref, dba_ref, dbv_ref)):
        db = db_ref[...]
        nd = jnp.sqrt(jnp.sum(db * db, axis=1, keepdims=True))
        dbn = db / (nd + 1e-8)
        s = lax.dot_general(q, dbn, (((1,), (1,)), ((), ())),
                            preferred_element_type=jnp.float32)
        s = jnp.where(valid, s, NEGF)
        for t in range(NGB):
            sims_ref[m, :, t] = s[:, t * G:(t + 1) * G].reshape(Q8, 8, G)
        gmv = jnp.max(s.reshape(Q, NGB, G), axis=2)
        gmp = jnp.concatenate(
            [gmv, jnp.full((Q, G - NGB), NEGINF, jnp.float32)], axis=1)
        gm_ref[m, :, 0] = gmp.reshape(Q8, 8, G)


def _tc_stage(query_feats, W_text, db_text, db_images, db_audio, db_video):
    return pl.pallas_call(
        _tc_body,
        grid=(NB,),
        in_specs=[
            pl.BlockSpec((Q, D_IN), lambda j: (0, 0)),
            pl.BlockSpec((D_IN, D), lambda j: (0, 0)),
            pl.BlockSpec((BK, D), lambda j: (j, 0)),
            pl.BlockSpec((BK, D), lambda j: (j, 0)),
            pl.BlockSpec((BK, D), lambda j: (j, 0)),
            pl.BlockSpec((BK, D), lambda j: (j, 0)),
        ],
        out_specs=[
            pl.BlockSpec((NM, Q8, NGB, 8, G), lambda j: (0, 0, j, 0, 0)),
            pl.BlockSpec((NM, Q8, 1, 8, G), lambda j: (0, 0, j, 0, 0)),
        ],
        out_shape=[
            jax.ShapeDtypeStruct((NM, Q8, NT, 8, G), jnp.float32),
            jax.ShapeDtypeStruct((NM, Q8, NB, 8, G), jnp.float32),
        ],
        scratch_shapes=[pltpu.VMEM((Q, D), jnp.float32)],
    )(query_feats, W_text, db_text, db_images, db_audio, db_video)


def _lane_best(i, carry, val_ref, id_ref):
    m, gi = carry
    v = val_ref[pl.ds(i * 16, 16)]
    g = id_ref[pl.ds(i * 16, 16)]
    upd = (v > m) | ((v == m) & (g < gi))
    return jnp.where(upd, v, m), jnp.where(upd, g, gi)


def _mask_out(i, _, val_ref, id_ref, vstar, istar):
    v = val_ref[pl.ds(i * 16, 16)]
    g = id_ref[pl.ds(i * 16, 16)]
    hit = (v == vstar) & (g == istar)
    val_ref[pl.ds(i * 16, 16)] = jnp.where(hit, NEGINF, v)
    return 0


def _select_round(val_ref, id_ref, ncv):
    m0 = jnp.full((16,), NEGINF, jnp.float32)
    g0 = jnp.full((16,), BIGI, jnp.int32)
    m, gi = lax.fori_loop(
        0, ncv, functools.partial(_lane_best, val_ref=val_ref, id_ref=id_ref),
        (m0, g0))
    vstar = jnp.max(m)
    istar = jnp.min(jnp.where(m == vstar, gi, BIGI))
    lax.fori_loop(
        0, ncv,
        functools.partial(_mask_out, val_ref=val_ref, id_ref=id_ref,
                          vstar=vstar, istar=istar), 0)
    return vstar, istar


def _compact(val_ref, id_ref, src_vec, id_vec, thresh, off):
    msk = src_vec >= thresh
    pos = off + plsc.cumsum(msk.astype(jnp.int32)) - 1
    plsc.store_scatter(val_ref, [pos], src_vec, mask=msk)
    plsc.store_scatter(id_ref, [pos], id_vec, mask=msk)
    return off + plsc.all_reduce_population_count(msk)


def _sc_bundle(bundle, gm_hbm, sims_hbm, vals_hbm, idx_hbm,
               gm_v, cval, cgid, l2_v, gsel_ref, outv, outi, sem, lanes):
    pltpu.sync_copy(gm_hbm.at[pl.ds(bundle * GMB, GMB)], gm_v)
    sims_base = bundle * (NT * 8 * G)

    def item_body(qr, _):
        def maxbody(j, mm):
            return jnp.maximum(mm, gm_v[pl.ds(j * (8 * G) + qr * G, 16)])
        t0 = jnp.min(lax.fori_loop(0, NB, maxbody,
                                   jnp.full((16,), NEGINF, jnp.float32)))

        def cbody(j, off):
            return _compact(cval, cgid, gm_v[pl.ds(j * (8 * G) + qr * G, 16)],
                            j * 16 + lanes, t0, off)
        off = lax.fori_loop(0, NB, cbody, jnp.zeros((16,), jnp.int32))
        cnt = jnp.max(off)
        plsc.store_scatter(cval, [cnt + lanes],
                           jnp.full((16,), NEGINF, jnp.float32))
        plsc.store_scatter(cgid, [cnt + lanes],
                           jnp.full((16,), BIGI, jnp.int32))
        ncv = (cnt + 15) >> 4

        base = sims_base + qr * G
        gsel = jnp.zeros((16,), jnp.int32)
        t1 = NEGINF
        copies = []
        for r in range(TOPK):
            vstar, istar = _select_round(cval, cgid, ncv)
            gsel = jnp.where(lanes == r, istar, gsel)
            t1 = vstar
            src = sims_hbm.at[pl.ds(base + istar * (8 * G), G)]
            copies.append(pltpu.async_copy(src, l2_v.at[pl.ds(r * G, G)],
                                           sem))
        for c in copies:
            c.wait()
        gsel_ref[...] = gsel

        def c2body(i, off2):
            p = i * 16 + lanes
            grp = plsc.load_gather(gsel_ref, [p >> 7])
            colv = (grp << 7) + (p & 127)
            return _compact(cval, cgid, l2_v[pl.ds(i * 16, 16)], colv, t1,
                            off2)
        off2 = lax.fori_loop(0, L2V, c2body, jnp.zeros((16,), jnp.int32))
        cnt2 = jnp.max(off2)
        plsc.store_scatter(cval, [cnt2 + lanes],
                           jnp.full((16,), NEGINF, jnp.float32))
        plsc.store_scatter(cgid, [cnt2 + lanes],
                           jnp.full((16,), BIGI, jnp.int32))
        ncv2 = (cnt2 + 15) >> 4

        ov = jnp.full((16,), 0.0, jnp.float32)
        oi = jnp.zeros((16,), jnp.int32)
        for r in range(TOPK):
            vstar, istar = _select_round(cval, cgid, ncv2)
            ov = jnp.where(lanes == r, vstar, ov)
            oi = jnp.where(lanes == r, istar, oi)
        outv[pl.ds(qr * TOPK, TOPK)] = ov
        outi[pl.ds(qr * TOPK, TOPK)] = oi
        return 0

    lax.fori_loop(0, 8, item_body, 0)
    pltpu.sync_copy(outv, vals_hbm.at[pl.ds(bundle * 8 * TOPK, 8 * TOPK)])
    pltpu.sync_copy(outi, idx_hbm.at[pl.ds(bundle * 8 * TOPK, 8 * TOPK)])
    return 0


def _sc_stage(gm_flat, sims_flat):
    mesh = plsc.VectorSubcoreMesh(core_axis_name="c", subcore_axis_name="s")

    @functools.partial(
        pl.kernel,
        out_type=[
            jax.ShapeDtypeStruct((ITEMS * TOPK,), jnp.float32),
            jax.ShapeDtypeStruct((ITEMS * TOPK,), jnp.int32),
        ],
        mesh=mesh,
        compiler_params=pltpu.CompilerParams(needs_layout_passes=False),
        scratch_types=[
            pltpu.VMEM((GMB,), jnp.float32),
            pltpu.VMEM((CBUF,), jnp.float32),
            pltpu.VMEM((CBUF,), jnp.int32),
            pltpu.VMEM((L2N,), jnp.float32),
            pltpu.VMEM((16,), jnp.int32),
            pltpu.VMEM((8 * TOPK,), jnp.float32),
            pltpu.VMEM((8 * TOPK,), jnp.int32),
            pltpu.SemaphoreType.DMA,
        ],
    )
    def sc_kernel(gm_hbm, sims_hbm, vals_hbm, idx_hbm,
                  gm_v, cval, cgid, l2_v, gsel_ref, outv, outi, sem):
        wid = lax.axis_index("s") * NC + lax.axis_index("c")
        lanes = jnp.arange(16, dtype=jnp.int32)
        lax.fori_loop(
            0, BPW,
            lambda b, _: _sc_bundle(wid * BPW + b, gm_hbm, sims_hbm,
                                    vals_hbm, idx_hbm, gm_v, cval, cgid,
                                    l2_v, gsel_ref, outv, outi, sem, lanes),
            0)

    return sc_kernel(gm_flat, sims_flat)


def kernel(query_feats, db_text, db_images, db_audio, db_video, W_text, top_k):
    sims, gm = _tc_stage(query_feats, W_text,
                         db_text, db_images, db_audio, db_video)
    vals_flat, idx_flat = _sc_stage(gm.reshape(NM * Q8 * NB * 8 * G),
                                    sims.reshape(NM * Q8 * NT * 8 * G))
    vals = vals_flat.reshape(NM, Q, TOPK)
    idx = idx_flat.reshape(NM, Q, TOPK)
    return (vals[0], idx[0], vals[1], idx[1],
            vals[2], idx[2], vals[3], idx[3])
